# baseline (speedup 1.0000x reference)
import dataclasses
import functools

import jax
import jax.numpy as jnp
from jax import lax
from jax.experimental import pallas as pl
from jax.experimental.pallas import tpu as pltpu
from jax.experimental.pallas import tpu_sc as plsc

N_NODES = 10000
N_EDGES = 320000
D_IN = 128
D_OUT = 128

NUM_CORES = 2
NUM_SUBCORES = 16
NUM_TILES = NUM_CORES * NUM_SUBCORES
LANES = 16

CHUNK = 80
CHUNKS_PER_TILE = 128
N_CHUNKS = NUM_TILES * CHUNKS_PER_TILE
E_PAD = N_CHUNKS * CHUNK
EDGES_PER_TILE = CHUNKS_PER_TILE * CHUNK
NBUF = 4
NROW = 8
ZBAND = 1000
NZ_TILES = N_NODES // ZBAND


def _matmul(x, w):
    def body(x_ref, w_ref, o_ref):
        o_ref[...] = jnp.dot(
            x_ref[...], w_ref[...], preferred_element_type=jnp.float32
        )

    return pl.pallas_call(
        body,
        out_shape=jax.ShapeDtypeStruct((N_NODES, D_OUT), jnp.float32),
    )(x, w)


def _sum_partials(p):
    def body(p_ref, o_ref):
        o_ref[...] = p_ref[0] + p_ref[1]

    return pl.pallas_call(
        body,
        out_shape=jax.ShapeDtypeStruct((N_NODES, D_OUT), jnp.float32),
    )(p)


def _sc_aggregate(support, row1d, col1d, val1d, zeros):
    mesh = plsc.VectorSubcoreMesh(
        core_axis_name="c",
        subcore_axis_name="s",
        num_cores=NUM_CORES,
        num_subcores=NUM_SUBCORES,
    )

    cp = pltpu.CompilerParams()
    if "needs_layout_passes" in pltpu.CompilerParams.__dataclass_fields__:
        cp = dataclasses.replace(cp, needs_layout_passes=False)
    if "use_tc_tiling_on_sc" in pltpu.CompilerParams.__dataclass_fields__:
        cp = dataclasses.replace(cp, use_tc_tiling_on_sc=False)

    @functools.partial(
        pl.kernel,
        out_type=jax.ShapeDtypeStruct(
            (NUM_CORES, NZ_TILES, ZBAND, D_OUT), jnp.float32
        ),
        mesh=mesh,
        compiler_params=cp,
        scratch_types=[
            pltpu.VMEM((NBUF, CHUNK), jnp.int32),
            pltpu.VMEM((NROW, CHUNK), jnp.int32),
            pltpu.VMEM((NBUF, CHUNK), jnp.float32),
            *[pltpu.VMEM((CHUNK, D_OUT), jnp.float32) for _ in range(NBUF)],
            pltpu.VMEM_SHARED((N_NODES, D_OUT), jnp.float32),
            pltpu.SemaphoreType.DMA((NBUF,)),
            pltpu.SemaphoreType.DMA((NBUF,)),
            pltpu.SemaphoreType.DMA((NROW,)),
        ],
    )
    def k(sup_hbm, row_hbm, col_hbm, val_hbm, zero_hbm, out_hbm,
          colr, rowr, valr, b0, b1, b2, b3, acc, gsem, ssem, isem):
        cid = lax.axis_index("c")
        sid = lax.axis_index("s")
        wid = sid * NUM_CORES + cid
        bufs = (b0, b1, b2, b3)
        ebase = wid * EDGES_PER_TILE

        @pl.when(sid < NZ_TILES)
        def _():
            pltpu.sync_copy(zero_hbm, acc.at[pl.ds(sid * ZBAND, ZBAND)])

        plsc.subcore_barrier()

        def idx_slices(t):
            sl = pl.ds(ebase + t * CHUNK, CHUNK)
            return sl

        def idx_dma_sync(t, s, sr):
            sl = idx_slices(t)
            pltpu.sync_copy(col_hbm.at[sl], colr.at[s])
            pltpu.sync_copy(row_hbm.at[sl], rowr.at[sr])
            pltpu.sync_copy(val_hbm.at[sl], valr.at[s])

        def idx_dma(t, s, sr, i):
            sl = idx_slices(t)
            pltpu.async_copy(col_hbm.at[sl], colr.at[s], isem.at[i])
            pltpu.async_copy(row_hbm.at[sl], rowr.at[sr], isem.at[i])
            pltpu.async_copy(val_hbm.at[sl], valr.at[s], isem.at[i])

        def wait_idx(t, s, sr, i):
            sl = idx_slices(t)
            pltpu.make_async_copy(col_hbm.at[sl], colr.at[s], isem.at[i]).wait()
            pltpu.make_async_copy(row_hbm.at[sl], rowr.at[sr], isem.at[i]).wait()
            pltpu.make_async_copy(val_hbm.at[sl], valr.at[s], isem.at[i]).wait()

        def gather(s, g):
            pltpu.async_copy(sup_hbm.at[colr.at[s]], bufs[g], gsem.at[g])

        def wait_gather(s, g):
            pltpu.make_async_copy(
                sup_hbm.at[colr.at[s]], bufs[g], gsem.at[g]
            ).wait()

        def scatter_add(sr, g):
            pltpu.async_copy(bufs[g], acc.at[rowr.at[sr]], ssem.at[g],
                             add=True)

        def wait_scatter(sr, g):
            pltpu.make_async_copy(
                bufs[g], acc.at[rowr.at[sr]], ssem.at[g]
            ).wait()

        for c in range(NBUF):
            idx_dma_sync(c, c, c)
        gather(0, 0)
        gather(1, 1)

        @pl.loop(0, CHUNKS_PER_TILE // NROW)
        def _(jo):
            for u in range(NROW):
                t = jo * NROW + u
                b = u % NBUF
                b2 = (u + 2) % NBUF
                r8 = u
                r4 = (u + 4) % NROW

                wait_gather(b, b)

                @pl.when(t + 2 < CHUNKS_PER_TILE)
                def _():
                    @pl.when(t + 2 >= NBUF)
                    def _():
                        wait_idx(t + 2, b2, (u + 2) % NROW, (u + 2) % NROW)

                    @pl.when(t >= 2)
                    def _():
                        wait_scatter((u + 6) % NROW, b2)

                    gather(b2, b2)

                rb = bufs[b]

                @pl.loop(0, CHUNK // LANES)
                def _(g2):
                    vals16 = valr[b, pl.ds(g2 * LANES, LANES)]
                    for e in range(LANES):
                        vsp = lax.gather(
                            vals16,
                            jnp.full((LANES, 1), e, jnp.int32),
                            dimension_numbers=lax.GatherDimensionNumbers(
                                offset_dims=(),
                                collapsed_slice_dims=(0,),
                                start_index_map=(0,),
                            ),
                            slice_sizes=(1,),
                            mode=lax.GatherScatterMode.PROMISE_IN_BOUNDS,
                        )
                        r = g2 * LANES + e
                        for q in range(D_OUT // LANES):
                            sl = pl.ds(q * LANES, LANES)
                            rb[r, sl] = rb[r, sl] * vsp

                @pl.when(t + 4 < CHUNKS_PER_TILE)
                def _():
                    idx_dma(t + 4, b, r4, r4)

                scatter_add(r8, b)

        for c in range(CHUNKS_PER_TILE - 4, CHUNKS_PER_TILE):
            wait_scatter(c % NROW, c % NBUF)
        plsc.subcore_barrier()

        @pl.when(sid < NZ_TILES)
        def _():
            pltpu.sync_copy(acc.at[pl.ds(sid * ZBAND, ZBAND)],
                            out_hbm.at[cid, sid])

    return k(support, row1d, col1d, val1d, zeros)


def kernel(edge_index, edge_values, input_feature, weight):
    support = _matmul(input_feature, weight)
    pad = E_PAD - N_EDGES
    spread = (jnp.arange(pad, dtype=jnp.int32) * 8) % N_NODES
    row1d = jnp.concatenate([edge_index[0].astype(jnp.int32), spread])
    col1d = jnp.concatenate([edge_index[1].astype(jnp.int32), spread])
    val1d = jnp.pad(edge_values, (0, pad))
    zeros = jnp.zeros((ZBAND, D_OUT), jnp.float32)
    partials = _sc_aggregate(support, row1d, col1d, val1d, zeros)
    partials = partials.reshape(NUM_CORES, N_NODES, D_OUT)
    return _sum_partials(partials)


# speedup vs baseline: 1.0025x; 1.0025x over previous
import dataclasses
import functools

import jax
import jax.numpy as jnp
from jax import lax
from jax.experimental import pallas as pl
from jax.experimental.pallas import tpu as pltpu
from jax.experimental.pallas import tpu_sc as plsc

N_NODES = 10000
N_EDGES = 320000
D_IN = 128
D_OUT = 128

NUM_CORES = 2
NUM_SUBCORES = 16
NUM_TILES = NUM_CORES * NUM_SUBCORES
LANES = 16

CHUNK = 80
CHUNKS_PER_TILE = 128
N_CHUNKS = NUM_TILES * CHUNKS_PER_TILE
E_PAD = N_CHUNKS * CHUNK
EDGES_PER_TILE = CHUNKS_PER_TILE * CHUNK
NBUF = 4
NROW = 8
ZBAND = 1000
NZ_TILES = N_NODES // ZBAND


def _matmul(x, w):
    def body(x_ref, w_ref, o_ref):
        o_ref[...] = jnp.dot(
            x_ref[...], w_ref[...], preferred_element_type=jnp.float32
        )

    return pl.pallas_call(
        body,
        out_shape=jax.ShapeDtypeStruct((N_NODES, D_OUT), jnp.float32),
    )(x, w)


def _sum_partials(p):
    def body(p_ref, o_ref):
        o_ref[...] = p_ref[0] + p_ref[1]

    return pl.pallas_call(
        body,
        out_shape=jax.ShapeDtypeStruct((N_NODES, D_OUT), jnp.float32),
    )(p)


def _sc_aggregate(support, row1d, col1d, val1d, zeros):
    mesh = plsc.VectorSubcoreMesh(
        core_axis_name="c",
        subcore_axis_name="s",
        num_cores=NUM_CORES,
        num_subcores=NUM_SUBCORES,
    )

    cp = pltpu.CompilerParams()
    if "needs_layout_passes" in pltpu.CompilerParams.__dataclass_fields__:
        cp = dataclasses.replace(cp, needs_layout_passes=False)

    @functools.partial(
        pl.kernel,
        out_type=jax.ShapeDtypeStruct(
            (NUM_CORES, NZ_TILES, ZBAND, D_OUT), jnp.float32
        ),
        mesh=mesh,
        compiler_params=cp,
        scratch_types=[
            pltpu.VMEM((NBUF, CHUNK), jnp.int32),
            pltpu.VMEM((NROW, CHUNK), jnp.int32),
            pltpu.VMEM((NBUF, CHUNK), jnp.float32),
            *[pltpu.VMEM((CHUNK, D_OUT), jnp.float32) for _ in range(NBUF)],
            pltpu.VMEM_SHARED((N_NODES, D_OUT), jnp.float32),
            pltpu.SemaphoreType.DMA((NBUF,)),
            pltpu.SemaphoreType.DMA((NBUF,)),
            pltpu.SemaphoreType.DMA((NROW,)),
        ],
    )
    def k(sup_hbm, row_hbm, col_hbm, val_hbm, zero_hbm, out_hbm,
          colr, rowr, valr, b0, b1, b2, b3, acc, gsem, ssem, isem):
        cid = lax.axis_index("c")
        sid = lax.axis_index("s")
        wid = sid * NUM_CORES + cid
        bufs = (b0, b1, b2, b3)
        ebase = wid * EDGES_PER_TILE

        @pl.when(sid < NZ_TILES)
        def _():
            pltpu.sync_copy(zero_hbm, acc.at[pl.ds(sid * ZBAND, ZBAND)])

        plsc.subcore_barrier()

        def idx_slices(t):
            sl = pl.ds(ebase + t * CHUNK, CHUNK)
            return sl

        def idx_dma_sync(t, s, sr):
            sl = idx_slices(t)
            pltpu.sync_copy(col_hbm.at[sl], colr.at[s])
            pltpu.sync_copy(row_hbm.at[sl], rowr.at[sr])
            pltpu.sync_copy(val_hbm.at[sl], valr.at[s])

        def idx_dma(t, s, sr, i):
            sl = idx_slices(t)
            pltpu.async_copy(col_hbm.at[sl], colr.at[s], isem.at[i])
            pltpu.async_copy(row_hbm.at[sl], rowr.at[sr], isem.at[i])
            pltpu.async_copy(val_hbm.at[sl], valr.at[s], isem.at[i])

        def wait_idx(t, s, sr, i):
            sl = idx_slices(t)
            pltpu.make_async_copy(col_hbm.at[sl], colr.at[s], isem.at[i]).wait()
            pltpu.make_async_copy(row_hbm.at[sl], rowr.at[sr], isem.at[i]).wait()
            pltpu.make_async_copy(val_hbm.at[sl], valr.at[s], isem.at[i]).wait()

        def gather(s, g):
            pltpu.async_copy(sup_hbm.at[colr.at[s]], bufs[g], gsem.at[g])

        def wait_gather(s, g):
            pltpu.make_async_copy(
                sup_hbm.at[colr.at[s]], bufs[g], gsem.at[g]
            ).wait()

        def scatter_add(sr, g):
            pltpu.async_copy(bufs[g], acc.at[rowr.at[sr]], ssem.at[g],
                             add=True)

        def wait_scatter(sr, g):
            pltpu.make_async_copy(
                bufs[g], acc.at[rowr.at[sr]], ssem.at[g]
            ).wait()

        for c in range(NBUF):
            idx_dma_sync(c, c, c)
        gather(0, 0)
        gather(1, 1)

        @pl.loop(0, CHUNKS_PER_TILE // NROW)
        def _(jo):
            for u in range(NROW):
                t = jo * NROW + u
                b = u % NBUF
                b2 = (u + 2) % NBUF
                r8 = u
                r4 = (u + 4) % NROW

                wait_gather(b, b)

                @pl.when(t + 2 < CHUNKS_PER_TILE)
                def _():
                    @pl.when(t + 2 >= NBUF)
                    def _():
                        wait_idx(t + 2, b2, (u + 2) % NROW, (u + 2) % NROW)

                    @pl.when(t >= 2)
                    def _():
                        wait_scatter((u + 6) % NROW, b2)

                    gather(b2, b2)

                rb = bufs[b]

                @pl.loop(0, CHUNK // LANES)
                def _(g2):
                    vals16 = valr[b, pl.ds(g2 * LANES, LANES)]
                    for e in range(LANES):
                        vsp = lax.gather(
                            vals16,
                            jnp.full((LANES, 1), e, jnp.int32),
                            dimension_numbers=lax.GatherDimensionNumbers(
                                offset_dims=(),
                                collapsed_slice_dims=(0,),
                                start_index_map=(0,),
                            ),
                            slice_sizes=(1,),
                            mode=lax.GatherScatterMode.PROMISE_IN_BOUNDS,
                        )
                        r = g2 * LANES + e
                        for q in range(D_OUT // LANES):
                            sl = pl.ds(q * LANES, LANES)
                            rb[r, sl] = rb[r, sl] * vsp

                @pl.when(t + 4 < CHUNKS_PER_TILE)
                def _():
                    idx_dma(t + 4, b, r4, r4)

                scatter_add(r8, b)

        for c in range(CHUNKS_PER_TILE - 4, CHUNKS_PER_TILE):
            wait_scatter(c % NROW, c % NBUF)
        plsc.subcore_barrier()

        @pl.when(sid < NZ_TILES)
        def _():
            pltpu.sync_copy(acc.at[pl.ds(sid * ZBAND, ZBAND)],
                            out_hbm.at[cid, sid])

    return k(support, row1d, col1d, val1d, zeros)


def kernel(edge_index, edge_values, input_feature, weight):
    support = _matmul(input_feature, weight)
    pad = E_PAD - N_EDGES
    spread = (jnp.arange(pad, dtype=jnp.int32) * 8) % N_NODES
    row1d = jnp.concatenate([edge_index[0].astype(jnp.int32), spread])
    col1d = jnp.concatenate([edge_index[1].astype(jnp.int32), spread])
    val1d = jnp.pad(edge_values, (0, pad))
    zeros = jnp.zeros((ZBAND, D_OUT), jnp.float32)
    partials = _sc_aggregate(support, row1d, col1d, val1d, zeros)
    partials = partials.reshape(NUM_CORES, N_NODES, D_OUT)
    return _sum_partials(partials)
